# baseline (speedup 1.0000x reference)
import functools

import jax
import jax.numpy as jnp
from jax import lax
from jax.experimental import pallas as pl
from jax.experimental.pallas import tpu as pltpu
from jax.experimental.pallas import tpu_sc as plsc

CHANS, SEQ, FEAT = 8, 4096, 1024
N_ROWS = SEQ // 4
BLOCK_S = 512
SEQ_BLOCKS = SEQ // BLOCK_S
LANES = 16


def _sc_scatter_body(idxs_hbm, warp_hbm, scale_hbm, idx_v, warp_v, scale_v):
    wid = lax.axis_index("s") * 2 + lax.axis_index("c")

    @pl.when(wid == 0)
    def _():
        pltpu.sync_copy(idxs_hbm, idx_v)
        pltpu.sync_copy(warp_hbm, warp_v)

        def init_body(i, _):
            scale_v[pl.ds(i * LANES, LANES)] = jnp.full((LANES,), 1.0, jnp.float32)
            return _

        lax.fori_loop(0, SEQ // LANES, init_body, 0)

        def scat_body(i, _):
            idx_chunk = idx_v[pl.ds(i * LANES, LANES)]
            w_chunk = warp_v[pl.ds(i * LANES, LANES)]
            plsc.store_scatter(scale_v, [idx_chunk], w_chunk)
            return _

        lax.fori_loop(0, N_ROWS // LANES, scat_body, 0)
        pltpu.sync_copy(scale_v, scale_hbm)


_sc_scatter = functools.partial(
    pl.kernel,
    out_type=jax.ShapeDtypeStruct((SEQ,), jnp.float32),
    mesh=plsc.VectorSubcoreMesh(core_axis_name="c", subcore_axis_name="s"),
    scratch_types=[
        pltpu.VMEM((N_ROWS,), jnp.int32),
        pltpu.VMEM((N_ROWS,), jnp.float32),
        pltpu.VMEM((SEQ,), jnp.float32),
    ],
    compiler_params=pltpu.CompilerParams(needs_layout_passes=False),
)(_sc_scatter_body)


def _tc_scale_body(scale_ref, x_ref, out_ref):
    out_ref[...] = x_ref[...] * scale_ref[...][None, :, :]


def kernel(x, idxs, warp):
    scale = _sc_scatter(idxs, warp.reshape(N_ROWS))
    scale2d = scale.reshape(SEQ, 1)
    return pl.pallas_call(
        _tc_scale_body,
        grid=(SEQ_BLOCKS, CHANS),
        in_specs=[
            pl.BlockSpec((BLOCK_S, 1), lambda s, c: (s, 0)),
            pl.BlockSpec((1, BLOCK_S, FEAT), lambda s, c: (c, s, 0)),
        ],
        out_specs=pl.BlockSpec((1, BLOCK_S, FEAT), lambda s, c: (c, s, 0)),
        out_shape=jax.ShapeDtypeStruct((CHANS, SEQ, FEAT), x.dtype),
        compiler_params=pltpu.CompilerParams(
            dimension_semantics=("arbitrary", "arbitrary"),
        ),
    )(scale2d, x)


# speedup vs baseline: 1.3826x; 1.3826x over previous
import jax
import jax.numpy as jnp
from jax.experimental import pallas as pl
from jax.experimental.pallas import tpu as pltpu

CHANS, SEQ, FEAT = 8, 4096, 1024
N_ROWS = SEQ // 4
BLOCK_S = 1024
SEQ_BLOCKS = SEQ // BLOCK_S


def _row_scale_body(idx_ref, warp_ref, x_ref, out_ref, scale_ref):
    c = pl.program_id(1)

    @pl.when(c == 0)
    def _compute_scale():
        s = pl.program_id(0)
        rows = jax.lax.broadcasted_iota(jnp.int32, (BLOCK_S, 1), 0) + s * BLOCK_S
        eq = rows == idx_ref[...]
        contrib = jnp.where(eq, warp_ref[...] - 1.0, 0.0)
        scale_ref[...] = 1.0 + jnp.sum(contrib, axis=1, keepdims=True)

    out_ref[...] = x_ref[...] * scale_ref[...][None, :, :]


def kernel(x, idxs, warp):
    idxs2d = idxs.reshape(1, N_ROWS)
    warp2d = warp.reshape(1, N_ROWS)
    return pl.pallas_call(
        _row_scale_body,
        grid=(SEQ_BLOCKS, CHANS),
        in_specs=[
            pl.BlockSpec((1, N_ROWS), lambda s, c: (0, 0)),
            pl.BlockSpec((1, N_ROWS), lambda s, c: (0, 0)),
            pl.BlockSpec((1, BLOCK_S, FEAT), lambda s, c: (c, s, 0)),
        ],
        out_specs=pl.BlockSpec((1, BLOCK_S, FEAT), lambda s, c: (c, s, 0)),
        out_shape=jax.ShapeDtypeStruct((CHANS, SEQ, FEAT), x.dtype),
        scratch_shapes=[pltpu.VMEM((BLOCK_S, 1), jnp.float32)],
        compiler_params=pltpu.CompilerParams(
            dimension_semantics=("arbitrary", "arbitrary"),
        ),
    )(idxs2d, warp2d, x)


# speedup vs baseline: 1.4170x; 1.0249x over previous
import jax
import jax.numpy as jnp
from jax.experimental import pallas as pl
from jax.experimental.pallas import tpu as pltpu

CHANS, SEQ, FEAT = 8, 4096, 1024
N_ROWS = SEQ // 4
BLOCK_S = 2048
SEQ_BLOCKS = SEQ // BLOCK_S


def _row_scale_body(idx_ref, warp_ref, x_ref, out_ref, scale_ref):
    c = pl.program_id(1)

    @pl.when(c == 0)
    def _compute_scale():
        s = pl.program_id(0)
        rows = jax.lax.broadcasted_iota(jnp.int32, (BLOCK_S, 1), 0) + s * BLOCK_S
        eq = rows == idx_ref[...]
        contrib = jnp.where(eq, warp_ref[...] - 1.0, 0.0)
        scale_ref[...] = 1.0 + jnp.sum(contrib, axis=1, keepdims=True)

    out_ref[...] = x_ref[...] * scale_ref[...][None, :, :]


def kernel(x, idxs, warp):
    idxs2d = idxs.reshape(1, N_ROWS)
    warp2d = warp.reshape(1, N_ROWS)
    return pl.pallas_call(
        _row_scale_body,
        grid=(SEQ_BLOCKS, CHANS),
        in_specs=[
            pl.BlockSpec((1, N_ROWS), lambda s, c: (0, 0)),
            pl.BlockSpec((1, N_ROWS), lambda s, c: (0, 0)),
            pl.BlockSpec((1, BLOCK_S, FEAT), lambda s, c: (c, s, 0)),
        ],
        out_specs=pl.BlockSpec((1, BLOCK_S, FEAT), lambda s, c: (c, s, 0)),
        out_shape=jax.ShapeDtypeStruct((CHANS, SEQ, FEAT), x.dtype),
        scratch_shapes=[pltpu.VMEM((BLOCK_S, 1), jnp.float32)],
        compiler_params=pltpu.CompilerParams(
            dimension_semantics=("arbitrary", "arbitrary"),
        ),
    )(idxs2d, warp2d, x)
